# baseline (speedup 1.0000x reference)
import functools

import jax
import jax.numpy as jnp
from jax import lax
from jax.experimental import pallas as pl
from jax.experimental.pallas import tpu as pltpu
from jax.experimental.pallas import tpu_sc as plsc


def _make_gather(N, V, D, NC, NS, chunk):
    NW = NC * NS
    n_per_w = N // NW
    n_chunks = n_per_w // chunk
    mesh = plsc.VectorSubcoreMesh(core_axis_name="c", subcore_axis_name="s")

    NBUF = 4
    LOOK = 2

    @functools.partial(
        pl.kernel,
        out_type=jax.ShapeDtypeStruct((N, D), jnp.float32),
        mesh=mesh,
        scratch_types=[
            pltpu.VMEM((n_per_w,), jnp.int32),
            pltpu.VMEM((NBUF, chunk, D), jnp.float32),
            [pltpu.SemaphoreType.DMA] * NBUF,
            [pltpu.SemaphoreType.DMA] * NBUF,
        ],
    )
    def gather_kernel(idx_hbm, table_hbm, out_hbm, idx_v, rows_v, gsems, ssems):
        sid = lax.axis_index("s")
        wid = sid * NC + lax.axis_index("c")
        base = wid * n_per_w
        pltpu.sync_copy(idx_hbm.at[pl.ds(base, n_per_w)], idx_v)

        def gather_start(i, b):
            off = pl.multiple_of(i * chunk, chunk)
            return pltpu.async_copy(
                table_hbm.at[idx_v.at[pl.ds(off, chunk)]], rows_v.at[b], gsems[b]
            )

        def gather_wait(i, b):
            off = pl.multiple_of(i * chunk, chunk)
            pltpu.make_async_copy(
                table_hbm.at[idx_v.at[pl.ds(off, chunk)]], rows_v.at[b], gsems[b]
            ).wait()

        def scatter_start(i, b):
            off = pl.multiple_of(i * chunk, chunk)
            return pltpu.async_copy(
                rows_v.at[b], out_hbm.at[pl.ds(base + off, chunk)], ssems[b]
            )

        def scatter_wait(i, b):
            off = pl.multiple_of(i * chunk, chunk)
            pltpu.make_async_copy(
                rows_v.at[b], out_hbm.at[pl.ds(base + off, chunk)], ssems[b]
            ).wait()

        gather_start(0, 0)
        gather_start(1, 1)
        for j in range(LOOK):
            gather_start(j + LOOK, (j + LOOK) % NBUF)
            gather_wait(j, j % NBUF)
            scatter_start(j, j % NBUF)

        @pl.loop(LOOK, n_chunks - LOOK, step=NBUF)
        def _(i):
            for t in range(NBUF):
                j = i + t
                b = (LOOK + t) % NBUF
                bp = (LOOK + t + LOOK) % NBUF
                scatter_wait(j - LOOK, bp)
                gather_start(j + LOOK, bp)
                gather_wait(j, b)
                scatter_start(j, b)

        for j in range(n_chunks - LOOK, n_chunks):
            gather_wait(j, j % NBUF)
            scatter_start(j, j % NBUF)
        for j in range(n_chunks - NBUF, n_chunks):
            scatter_wait(j, j % NBUF)

    return gather_kernel


def _make_rowdma(N, V, D, NC, NS):
    NW = NC * NS
    n_per_w = N // NW
    G = 16
    mesh = plsc.VectorSubcoreMesh(core_axis_name="c", subcore_axis_name="s")

    @functools.partial(
        pl.kernel,
        out_type=jax.ShapeDtypeStruct((N * D,), jnp.float32),
        mesh=mesh,
        scratch_types=[
            pltpu.VMEM_SHARED((V * D,), jnp.float32),
            pltpu.VMEM((n_per_w,), jnp.int32),
            pltpu.SemaphoreType.DMA,
        ],
    )
    def rowdma_kernel(idx_hbm, table_hbm, out_hbm, table_sp, idx_v, ssem):
        sid = lax.axis_index("s")
        wid = sid * NC + lax.axis_index("c")
        base = wid * n_per_w

        tchunk = (V * D) // NS
        toff = pl.multiple_of(sid * tchunk, 8)
        pltpu.sync_copy(
            table_hbm.at[pl.ds(toff, tchunk)], table_sp.at[pl.ds(toff, tchunk)]
        )
        pltpu.sync_copy(idx_hbm.at[pl.ds(base, n_per_w)], idx_v)
        plsc.subcore_barrier()

        def fire(i0):
            v = idx_v[pl.ds(i0, G)]
            for t in range(G):
                src = pl.multiple_of(v[t] * D, 8)
                dst = pl.multiple_of((base + i0 + t) * D, 8)
                pltpu.async_copy(
                    table_sp.at[pl.ds(src, D)], out_hbm.at[pl.ds(dst, D)], ssem
                )

        def drain():
            for _ in range(G):
                pltpu.make_async_copy(
                    table_sp.at[pl.ds(0, D)], out_hbm.at[pl.ds(base * D, D)], ssem
                ).wait()

        fire(0)

        @pl.loop(G, n_per_w, step=G)
        def _(i0):
            fire(pl.multiple_of(i0, G))
            drain()

        drain()

    return rowdma_kernel


def kernel(positions, position_embeddings):
    B, S = positions.shape
    V, D = position_embeddings.shape
    N = B * S
    info = plsc.get_sparse_core_info()
    fn = _make_rowdma(N, V, D, info.num_cores, info.num_subcores)
    out = fn(
        positions.reshape(N).astype(jnp.int32),
        position_embeddings.reshape(V * D),
    )
    return out.reshape(B, S, D)


# speedup vs baseline: 1.1914x; 1.1914x over previous
import functools

import jax
import jax.numpy as jnp
from jax import lax
from jax.experimental import pallas as pl
from jax.experimental.pallas import tpu as pltpu
from jax.experimental.pallas import tpu_sc as plsc


def _make_gather(N, V, D, NC, NS, chunk):
    NW = NC * NS
    n_per_w = N // NW
    n_chunks = n_per_w // chunk
    mesh = plsc.VectorSubcoreMesh(core_axis_name="c", subcore_axis_name="s")

    NBUF = 4
    LOOK = 2

    @functools.partial(
        pl.kernel,
        out_type=jax.ShapeDtypeStruct((N, D), jnp.float32),
        mesh=mesh,
        scratch_types=[
            pltpu.VMEM((n_per_w,), jnp.int32),
            pltpu.VMEM((NBUF, chunk, D), jnp.float32),
            [pltpu.SemaphoreType.DMA] * NBUF,
            [pltpu.SemaphoreType.DMA] * NBUF,
        ],
    )
    def gather_kernel(idx_hbm, table_hbm, out_hbm, idx_v, rows_v, gsems, ssems):
        sid = lax.axis_index("s")
        wid = sid * NC + lax.axis_index("c")
        base = wid * n_per_w
        pltpu.sync_copy(idx_hbm.at[pl.ds(base, n_per_w)], idx_v)

        def gather_start(i, b):
            off = pl.multiple_of(i * chunk, chunk)
            return pltpu.async_copy(
                table_hbm.at[idx_v.at[pl.ds(off, chunk)]], rows_v.at[b], gsems[b]
            )

        def gather_wait(i, b):
            off = pl.multiple_of(i * chunk, chunk)
            pltpu.make_async_copy(
                table_hbm.at[idx_v.at[pl.ds(off, chunk)]], rows_v.at[b], gsems[b]
            ).wait()

        def scatter_start(i, b):
            off = pl.multiple_of(i * chunk, chunk)
            return pltpu.async_copy(
                rows_v.at[b], out_hbm.at[pl.ds(base + off, chunk)], ssems[b]
            )

        def scatter_wait(i, b):
            off = pl.multiple_of(i * chunk, chunk)
            pltpu.make_async_copy(
                rows_v.at[b], out_hbm.at[pl.ds(base + off, chunk)], ssems[b]
            ).wait()

        gather_start(0, 0)
        gather_start(1, 1)
        for j in range(LOOK):
            gather_start(j + LOOK, (j + LOOK) % NBUF)
            gather_wait(j, j % NBUF)
            scatter_start(j, j % NBUF)

        @pl.loop(LOOK, n_chunks - LOOK, step=NBUF)
        def _(i):
            for t in range(NBUF):
                j = i + t
                b = (LOOK + t) % NBUF
                bp = (LOOK + t + LOOK) % NBUF
                scatter_wait(j - LOOK, bp)
                gather_start(j + LOOK, bp)
                gather_wait(j, b)
                scatter_start(j, b)

        for j in range(n_chunks - LOOK, n_chunks):
            gather_wait(j, j % NBUF)
            scatter_start(j, j % NBUF)
        for j in range(n_chunks - NBUF, n_chunks):
            scatter_wait(j, j % NBUF)

    return gather_kernel


def _make_rowdma(N, V, D, NC, NS):
    NW = NC * NS
    n_per_w = N // NW
    G = 16
    mesh = plsc.VectorSubcoreMesh(core_axis_name="c", subcore_axis_name="s")

    @functools.partial(
        pl.kernel,
        out_type=jax.ShapeDtypeStruct((N * D,), jnp.float32),
        mesh=mesh,
        scratch_types=[
            pltpu.VMEM_SHARED((V * D,), jnp.float32),
            pltpu.VMEM((n_per_w,), jnp.int32),
            pltpu.SemaphoreType.DMA,
        ],
    )
    def rowdma_kernel(idx_hbm, table_hbm, out_hbm, table_sp, idx_v, ssem):
        sid = lax.axis_index("s")
        wid = sid * NC + lax.axis_index("c")
        base = wid * n_per_w

        tchunk = (V * D) // NS
        toff = pl.multiple_of(sid * tchunk, 8)
        pltpu.sync_copy(
            table_hbm.at[pl.ds(toff, tchunk)], table_sp.at[pl.ds(toff, tchunk)]
        )
        pltpu.sync_copy(idx_hbm.at[pl.ds(base, n_per_w)], idx_v)
        plsc.subcore_barrier()

        def fire(i0):
            v = idx_v[pl.ds(i0, G)]
            for t in range(G):
                src = pl.multiple_of(v[t] * D, 8)
                dst = pl.multiple_of((base + i0 + t) * D, 8)
                pltpu.async_copy(
                    table_sp.at[pl.ds(src, D)], out_hbm.at[pl.ds(dst, D)], ssem
                )

        def drain():
            for _ in range(G):
                pltpu.make_async_copy(
                    table_sp.at[pl.ds(0, D)], out_hbm.at[pl.ds(base * D, D)], ssem
                ).wait()

        fire(0)

        @pl.loop(G, n_per_w, step=G)
        def _(i0):
            fire(pl.multiple_of(i0, G))
            drain()

        drain()

    return rowdma_kernel


def _make_quarter(N, V, D, NC, NS):
    NW = NC * NS
    NQ = 4
    NSEG = NW // NQ
    vq = V // NQ
    n_seg = N // NSEG
    SUB = 4608
    n_sub = n_seg // SUB
    mesh = plsc.VectorSubcoreMesh(core_axis_name="c", subcore_axis_name="s")

    @functools.partial(
        pl.kernel,
        out_type=jax.ShapeDtypeStruct((N * D,), jnp.float32),
        mesh=mesh,
        scratch_types=[
            pltpu.VMEM((vq * D,), jnp.float32),
            pltpu.VMEM((SUB,), jnp.int32),
            pltpu.SemaphoreType.DMA,
        ],
    )
    def quarter_kernel(idx_hbm, table_hbm, out_hbm, slab, idx_v, ssem):
        wid = lax.axis_index("s") * NC + lax.axis_index("c")
        q = lax.rem(wid, NQ)
        seg = lax.div(wid, NQ)
        qlo = q * vq
        slab_off = pl.multiple_of(q * (vq * D), 8)
        pltpu.sync_copy(table_hbm.at[pl.ds(slab_off, vq * D)], slab)
        seg_base = seg * n_seg

        def drain_one(_):
            pltpu.make_async_copy(
                slab.at[pl.ds(0, D)],
                out_hbm.at[pl.ds(pl.multiple_of(seg_base * D, 8), D)],
                ssem,
            ).wait()

        @pl.loop(0, n_sub)
        def _(c):
            sub_base = seg_base + c * SUB
            pltpu.sync_copy(
                idx_hbm.at[pl.ds(pl.multiple_of(sub_base, 8), SUB)], idx_v
            )

            @pl.loop(0, SUB // 16, init_carry=jnp.int32(0))
            def fired(g, cnt):
                o = pl.multiple_of(g * 16, 16)
                v = idx_v[pl.ds(o, 16)]
                for t in range(16):
                    p = v[t]
                    m = jnp.logical_and(p >= qlo, p < qlo + vq)

                    @pl.when(m)
                    def _():
                        src = pl.multiple_of((p - qlo) * D, 8)
                        dst = pl.multiple_of((sub_base + o + t) * D, 8)
                        pltpu.async_copy(
                            slab.at[pl.ds(src, D)],
                            out_hbm.at[pl.ds(dst, D)],
                            ssem,
                        )

                    cnt = cnt + jnp.where(m, jnp.int32(1), jnp.int32(0))
                return cnt

            pl.loop(0, fired)(drain_one)

    return quarter_kernel


def kernel(positions, position_embeddings):
    B, S = positions.shape
    V, D = position_embeddings.shape
    N = B * S
    info = plsc.get_sparse_core_info()
    fn = _make_quarter(N, V, D, info.num_cores, info.num_subcores)
    out = fn(
        positions.reshape(N).astype(jnp.int32),
        position_embeddings.reshape(V * D),
    )
    return out.reshape(B, S, D)


# speedup vs baseline: 2.0610x; 1.7299x over previous
import functools

import jax
import jax.numpy as jnp
from jax import lax
from jax.experimental import pallas as pl
from jax.experimental.pallas import tpu as pltpu
from jax.experimental.pallas import tpu_sc as plsc


def _make_gather(N, V, D, NC, NS, chunk):
    NW = NC * NS
    n_per_w = N // NW
    n_chunks = n_per_w // chunk
    mesh = plsc.VectorSubcoreMesh(core_axis_name="c", subcore_axis_name="s")

    NBUF = 2
    LOOK = 1

    @functools.partial(
        pl.kernel,
        out_type=jax.ShapeDtypeStruct((N, D), jnp.float32),
        mesh=mesh,
        scratch_types=[
            pltpu.VMEM((n_per_w,), jnp.int32),
            pltpu.VMEM((NBUF, chunk, D), jnp.float32),
            [pltpu.SemaphoreType.DMA] * NBUF,
            [pltpu.SemaphoreType.DMA] * NBUF,
        ],
    )
    def gather_kernel(idx_hbm, table_hbm, out_hbm, idx_v, rows_v, gsems, ssems):
        sid = lax.axis_index("s")
        wid = sid * NC + lax.axis_index("c")
        base = wid * n_per_w
        pltpu.sync_copy(idx_hbm.at[pl.ds(base, n_per_w)], idx_v)

        def gather_start(i, b):
            off = pl.multiple_of(i * chunk, chunk)
            return pltpu.async_copy(
                table_hbm.at[idx_v.at[pl.ds(off, chunk)]], rows_v.at[b], gsems[b]
            )

        def gather_wait(i, b):
            off = pl.multiple_of(i * chunk, chunk)
            pltpu.make_async_copy(
                table_hbm.at[idx_v.at[pl.ds(off, chunk)]], rows_v.at[b], gsems[b]
            ).wait()

        def scatter_start(i, b):
            off = pl.multiple_of(i * chunk, chunk)
            return pltpu.async_copy(
                rows_v.at[b], out_hbm.at[pl.ds(base + off, chunk)], ssems[b]
            )

        def scatter_wait(i, b):
            off = pl.multiple_of(i * chunk, chunk)
            pltpu.make_async_copy(
                rows_v.at[b], out_hbm.at[pl.ds(base + off, chunk)], ssems[b]
            ).wait()

        for j in range(LOOK):
            gather_start(j, j % NBUF)
        for j in range(LOOK):
            gather_start(j + LOOK, (j + LOOK) % NBUF)
            gather_wait(j, j % NBUF)
            scatter_start(j, j % NBUF)

        @pl.loop(LOOK, n_chunks - LOOK, step=NBUF)
        def _(i):
            for t in range(NBUF):
                j = i + t
                b = (LOOK + t) % NBUF
                bp = (LOOK + t + LOOK) % NBUF
                scatter_wait(j - LOOK, bp)
                gather_start(j + LOOK, bp)
                gather_wait(j, b)
                scatter_start(j, b)

        for j in range(n_chunks - LOOK, n_chunks):
            gather_wait(j, j % NBUF)
            scatter_start(j, j % NBUF)
        for j in range(n_chunks - NBUF, n_chunks):
            scatter_wait(j, j % NBUF)

    return gather_kernel


def kernel(positions, position_embeddings):
    B, S = positions.shape
    V, D = position_embeddings.shape
    N = B * S
    info = plsc.get_sparse_core_info()
    fn = _make_gather(N, V, D, info.num_cores, info.num_subcores, 72)
    out = fn(positions.reshape(N).astype(jnp.int32), position_embeddings)
    return out.reshape(B, S, D)


# speedup vs baseline: 2.0611x; 1.0001x over previous
import functools

import jax
import jax.numpy as jnp
from jax import lax
from jax.experimental import pallas as pl
from jax.experimental.pallas import tpu as pltpu
from jax.experimental.pallas import tpu_sc as plsc


def _make_gather(N, V, D, NC, NS, chunk):
    NW = NC * NS
    n_per_w = N // NW
    n_chunks = n_per_w // chunk
    mesh = plsc.VectorSubcoreMesh(core_axis_name="c", subcore_axis_name="s")

    NBUF = 4
    LOOK = 2

    @functools.partial(
        pl.kernel,
        out_type=jax.ShapeDtypeStruct((N, D), jnp.float32),
        mesh=mesh,
        scratch_types=[
            pltpu.VMEM((n_per_w,), jnp.int32),
            pltpu.VMEM((NBUF, chunk, D), jnp.float32),
            [pltpu.SemaphoreType.DMA] * NBUF,
            [pltpu.SemaphoreType.DMA] * NBUF,
        ],
    )
    def gather_kernel(idx_hbm, table_hbm, out_hbm, idx_v, rows_v, gsems, ssems):
        sid = lax.axis_index("s")
        wid = sid * NC + lax.axis_index("c")
        base = wid * n_per_w
        pltpu.sync_copy(idx_hbm.at[pl.ds(base, n_per_w)], idx_v)

        def gather_start(i, b):
            off = pl.multiple_of(i * chunk, chunk)
            return pltpu.async_copy(
                table_hbm.at[idx_v.at[pl.ds(off, chunk)]], rows_v.at[b], gsems[b]
            )

        def gather_wait(i, b):
            off = pl.multiple_of(i * chunk, chunk)
            pltpu.make_async_copy(
                table_hbm.at[idx_v.at[pl.ds(off, chunk)]], rows_v.at[b], gsems[b]
            ).wait()

        def scatter_start(i, b):
            off = pl.multiple_of(i * chunk, chunk)
            return pltpu.async_copy(
                rows_v.at[b], out_hbm.at[pl.ds(base + off, chunk)], ssems[b]
            )

        def scatter_wait(i, b):
            off = pl.multiple_of(i * chunk, chunk)
            pltpu.make_async_copy(
                rows_v.at[b], out_hbm.at[pl.ds(base + off, chunk)], ssems[b]
            ).wait()

        gather_start(0, 0)
        gather_start(1, 1)
        for j in range(LOOK):
            gather_start(j + LOOK, (j + LOOK) % NBUF)
            gather_wait(j, j % NBUF)
            scatter_start(j, j % NBUF)

        @pl.loop(LOOK, n_chunks - LOOK, step=NBUF)
        def _(i):
            for t in range(NBUF):
                j = i + t
                b = (LOOK + t) % NBUF
                bp = (LOOK + t + LOOK) % NBUF
                scatter_wait(j - LOOK, bp)
                gather_start(j + LOOK, bp)
                gather_wait(j, b)
                scatter_start(j, b)

        for j in range(n_chunks - LOOK, n_chunks):
            gather_wait(j, j % NBUF)
            scatter_start(j, j % NBUF)
        for j in range(n_chunks - NBUF, n_chunks):
            scatter_wait(j, j % NBUF)

    return gather_kernel


def kernel(positions, position_embeddings):
    B, S = positions.shape
    V, D = position_embeddings.shape
    N = B * S
    info = plsc.get_sparse_core_info()
    fn = _make_gather(N, V, D, info.num_cores, info.num_subcores, 32)
    out = fn(positions.reshape(N).astype(jnp.int32), position_embeddings)
    return out.reshape(B, S, D)
